# Initial kernel scaffold; baseline (speedup 1.0000x reference)
import jax
import jax.numpy as jnp
from jax.experimental import pallas as pl


def kernel(x, edge_list, batch, W1, b1, p, W3, b3, Wfc, bfc):
    raise NotImplementedError("write your pallas kernel here")


# speedup vs baseline: 44.1083x; 44.1083x over previous
import functools

import jax
import jax.numpy as jnp
from jax import lax
from jax.experimental import pallas as pl
from jax.experimental.pallas import tpu as pltpu
from jax.experimental.pallas import tpu_sc as plsc

N = 10000
E = 320000
F_IN = 128
HID = 32
G = 16
K = N // 2

NC = 2
NS = 16
NW = NC * NS
NP = 10240
RPT = NP // NS
BB = 80
EPW = E // NW
NBPW = EPW // BB
CW = 16

_f32 = jnp.float32
_i32 = jnp.int32


def _zero_stage(stage, rows, width):
    z = jnp.zeros((16,), _f32)

    def zrow(i, t):
        for w0 in range(0, width, 16):
            stage[i, pl.ds(w0, 16)] = z
        return t

    lax.fori_loop(0, rows, zrow, 0)


@functools.lru_cache(maxsize=None)
def _sc_kernels():
    mesh = plsc.VectorSubcoreMesh(
        core_axis_name="c", subcore_axis_name="s",
        num_cores=NC, num_subcores=NS,
    )
    cparams = pltpu.CompilerParams(use_tc_tiling_on_sc=False)

    @functools.partial(
        pl.kernel,
        out_type=jax.ShapeDtypeStruct((NC, NP, HID), _f32),
        mesh=mesh,
        scratch_types=[
            pltpu.VMEM_SHARED((NP, HID), _f32),
            pltpu.VMEM((NBPW, BB), _i32),
            pltpu.VMEM((NBPW, BB), _i32),
            pltpu.VMEM((BB, HID), _f32),
            pltpu.VMEM((RPT, HID), _f32),
            pltpu.SemaphoreType.DMA,
        ],
        compiler_params=cparams,
    )
    def sc_feat(src3d, dst3d, g, out, acc, sidx, didx, rows, stage, sem):
        c = lax.axis_index("c")
        s = lax.axis_index("s")
        wid = c * NS + s
        _zero_stage(stage, RPT, HID)
        pltpu.sync_copy(stage, acc.at[pl.ds(s * RPT, RPT)])
        plsc.subcore_barrier()
        pltpu.sync_copy(src3d.at[wid], sidx)
        pltpu.sync_copy(dst3d.at[wid], didx)

        def step(j, t):
            pltpu.async_copy(g.at[sidx.at[j]], rows, sem).wait()
            pltpu.sync_copy(rows, acc.at[didx.at[j]], add=True)
            return t

        lax.fori_loop(0, NBPW, step, 0)
        plsc.subcore_barrier()
        pltpu.sync_copy(acc.at[pl.ds(s * RPT, RPT)], stage)
        pltpu.sync_copy(stage, out.at[c, pl.ds(s * RPT, RPT)])

    @functools.partial(
        pl.kernel,
        out_type=jax.ShapeDtypeStruct((NC, NP, CW), _f32),
        mesh=mesh,
        scratch_types=[
            pltpu.VMEM_SHARED((NP, CW), _f32),
            pltpu.VMEM((NBPW, BB), _i32),
            pltpu.VMEM((BB, CW), _f32),
            pltpu.VMEM((RPT, CW), _f32),
        ],
        compiler_params=cparams,
    )
    def sc_cnt1(dst3d, out, acc, didx, vals, stage):
        c = lax.axis_index("c")
        s = lax.axis_index("s")
        wid = c * NS + s
        _zero_stage(stage, RPT, CW)
        pltpu.sync_copy(stage, acc.at[pl.ds(s * RPT, RPT)])
        one = jnp.ones((16,), _f32)

        def orow(i, t):
            vals[i, pl.ds(0, 16)] = one
            return t

        lax.fori_loop(0, BB, orow, 0)
        plsc.subcore_barrier()
        pltpu.sync_copy(dst3d.at[wid], didx)

        def step(j, t):
            pltpu.sync_copy(vals, acc.at[didx.at[j]], add=True)
            return t

        lax.fori_loop(0, NBPW, step, 0)
        plsc.subcore_barrier()
        pltpu.sync_copy(acc.at[pl.ds(s * RPT, RPT)], stage)
        pltpu.sync_copy(stage, out.at[c, pl.ds(s * RPT, RPT)])

    @functools.partial(
        pl.kernel,
        out_type=jax.ShapeDtypeStruct((NC, NP, CW), _f32),
        mesh=mesh,
        scratch_types=[
            pltpu.VMEM_SHARED((NP, CW), _f32),
            pltpu.VMEM((NBPW, BB), _i32),
            pltpu.VMEM((NBPW, BB), _i32),
            pltpu.VMEM((BB, CW), _f32),
            pltpu.VMEM((RPT, CW), _f32),
            pltpu.SemaphoreType.DMA,
        ],
        compiler_params=cparams,
    )
    def sc_cnt2(src3d, dst3d, sel16, out, acc, sidx, didx, vals, stage, sem):
        c = lax.axis_index("c")
        s = lax.axis_index("s")
        wid = c * NS + s
        _zero_stage(stage, RPT, CW)
        pltpu.sync_copy(stage, acc.at[pl.ds(s * RPT, RPT)])
        plsc.subcore_barrier()
        pltpu.sync_copy(src3d.at[wid], sidx)
        pltpu.sync_copy(dst3d.at[wid], didx)

        def step(j, t):
            pltpu.async_copy(sel16.at[sidx.at[j]], vals, sem).wait()
            pltpu.sync_copy(vals, acc.at[didx.at[j]], add=True)
            return t

        lax.fori_loop(0, NBPW, step, 0)
        plsc.subcore_barrier()
        pltpu.sync_copy(acc.at[pl.ds(s * RPT, RPT)], stage)
        pltpu.sync_copy(stage, out.at[c, pl.ds(s * RPT, RPT)])

    return sc_feat, sc_cnt1, sc_cnt2


def _tc_pre_body(x_ref, w1_ref, b1_ref, cntp_ref, g1_ref, self_ref):
    h1 = jnp.dot(x_ref[...], w1_ref[...], preferred_element_type=_f32)
    cnt = cntp_ref[0, :, 0:1] + cntp_ref[1, :, 0:1]
    d1 = cnt + 1.0
    g1_ref[...] = h1 * lax.rsqrt(d1)
    self_ref[...] = h1 / d1 + b1_ref[...]


_tc_pre = pl.pallas_call(
    _tc_pre_body,
    out_shape=(
        jax.ShapeDtypeStruct((NP, HID), _f32),
        jax.ShapeDtypeStruct((NP, HID), _f32),
    ),
)

def _tc_mid_body(accp, cntp, selft, p2, w3, sel16_o, h2t_o):
    _MININT = jnp.int32(-2147483648)
    _MAXINT = jnp.int32(2147483647)
    raw = accp[0] + accp[1]
    cnt = cntp[0, :, 0:1] + cntp[1, :, 0:1]
    dinv = lax.rsqrt(cnt + 1.0)
    h = jnp.maximum(raw * dinv + selft[...], 0.0)
    pv = p2[...]
    pn = lax.rsqrt(jnp.sum(pv * pv))
    score = jnp.dot(h, pv, preferred_element_type=_f32) * pn
    rowid = lax.broadcasted_iota(_i32, (NP, 1), 0)
    score = jnp.where(rowid < N, score, -jnp.inf)
    b = lax.bitcast_convert_type(score, _i32)
    ks = b ^ (_MAXINT & (b >> 31))
    def bstep(i, prefix):
        cand = prefix | jnp.left_shift(jnp.int32(1), 31 - i)
        c = jnp.sum((ks >= (cand ^ _MININT)).astype(_i32))
        return jnp.where(c >= K, cand, prefix)

    prefix = lax.fori_loop(0, 32, bstep, jnp.int32(0))
    sel = (ks >= (prefix ^ _MININT)).astype(_f32)
    sel16_o[...] = jnp.broadcast_to(sel, (NP, CW))
    xpm = h * (jnp.tanh(score) * sel)
    h2t_o[...] = jnp.dot(xpm, w3[...], preferred_element_type=_f32)


_tc_mid = pl.pallas_call(
    _tc_mid_body,
    out_shape=(
        jax.ShapeDtypeStruct((NP, CW), _f32),
        jax.ShapeDtypeStruct((NP, HID), _f32),
    ),
)


def _tc_g2_body(cntp, h2t, sel16, g2_o):
    cnt = cntp[0, :, 0:1] + cntp[1, :, 0:1]
    g2_o[...] = h2t[...] * lax.rsqrt(cnt + 1.0) * sel16[:, 0:1]


_tc_g2 = pl.pallas_call(
    _tc_g2_body, out_shape=jax.ShapeDtypeStruct((NP, HID), _f32)
)


def _tc_post_body(accp, cntp, h2t, sel16, batchb, b3_ref, wfc, bfc, out_o):
    raw = accp[0] + accp[1]
    cnt = cntp[0, :, 0:1] + cntp[1, :, 0:1]
    d2 = cnt + 1.0
    h2 = jnp.maximum(raw * lax.rsqrt(d2) + h2t[...] / d2 + b3_ref[...], 0.0)
    sel = sel16[:, 0:1]
    h2m = h2 * sel
    gid = lax.broadcasted_iota(_i32, (G, NP), 0)
    oh = (gid == batchb[...]).astype(_f32)
    sums = jnp.dot(oh, h2m, preferred_element_type=_f32)
    counts = jnp.dot(oh, sel, preferred_element_type=_f32)
    mean = sums / jnp.maximum(counts, 1.0)
    z = jnp.dot(mean, wfc[...], preferred_element_type=_f32) + bfc[...]
    out_o[...] = 1.0 / (1.0 + jnp.exp(-z))


_tc_post = pl.pallas_call(
    _tc_post_body, out_shape=jax.ShapeDtypeStruct((G, 1), _f32)
)


def kernel(x, edge_list, batch, W1, b1, p, W3, b3, Wfc, bfc):
    src3d = edge_list[:, 0].astype(_i32).reshape(NW, NBPW, BB)
    dst3d = edge_list[:, 1].astype(_i32).reshape(NW, NBPW, BB)
    xpad = jnp.pad(x.astype(_f32), ((0, NP - N), (0, 0)))
    batchb = jnp.pad(
        batch.astype(_i32), (0, NP - N), constant_values=G
    ).reshape(1, NP)
    b1r = jnp.reshape(b1, (1, HID)).astype(_f32)
    b3r = jnp.reshape(b3, (1, HID)).astype(_f32)
    p2 = jnp.reshape(p, (HID, 1)).astype(_f32)
    wfc = jnp.reshape(Wfc, (HID, 1)).astype(_f32)
    bfcr = jnp.reshape(bfc, (1, 1)).astype(_f32)

    sc_feat, sc_cnt1, sc_cnt2 = _sc_kernels()
    cnt1p = sc_cnt1(dst3d)
    g1, selft = _tc_pre(xpad, W1.astype(_f32), b1r, cnt1p)
    acc1p = sc_feat(src3d, dst3d, g1)
    sel16, h2t = _tc_mid(acc1p, cnt1p, selft, p2, W3.astype(_f32))
    cnt2p = sc_cnt2(src3d, dst3d, sel16)
    g2 = _tc_g2(cnt2p, h2t, sel16)
    acc2p = sc_feat(src3d, dst3d, g2)
    out = _tc_post(acc2p, cnt2p, h2t, sel16, batchb, b3r, wfc, bfcr)
    return out.reshape(G)
